# Initial kernel scaffold; baseline (speedup 1.0000x reference)
import jax
import jax.numpy as jnp
from jax.experimental import pallas as pl


def kernel(x, table):
    raise NotImplementedError("write your pallas kernel here")


# speedup vs baseline: 1.5597x; 1.5597x over previous
import functools

import jax
import jax.numpy as jnp
from jax import lax
from jax.experimental import pallas as pl
from jax.experimental.pallas import tpu as pltpu
from jax.experimental.pallas import tpu_sc as plsc

BATCH = 16384
FIELDS = 26
HIDDEN = 32
TOTAL = BATCH * FIELDS

NC = 2
NS = 16
NW = NC * NS
PER_W = TOTAL // NW
CHUNK = 128
G = PER_W // CHUNK
NBUF = 8
GROUPS = G // NBUF
GROUP_ROWS = NBUF * CHUNK

_mesh = plsc.VectorSubcoreMesh(core_axis_name="c", subcore_axis_name="s")


@functools.partial(
    pl.kernel,
    out_type=jax.ShapeDtypeStruct((TOTAL, HIDDEN), jnp.float32),
    mesh=_mesh,
    scratch_types=[
        pltpu.VMEM((G, CHUNK), jnp.int32),
        pltpu.VMEM((GROUP_ROWS, HIDDEN), jnp.float32),
        pltpu.SemaphoreType.DMA,
    ],
    compiler_params=pltpu.CompilerParams(use_tc_tiling_on_sc=False),
)
def _sc_gather(idx_hbm, table_hbm, out_hbm, idx_v, rows_v, sem):
    wid = lax.axis_index("s") * NC + lax.axis_index("c")
    base = wid * PER_W
    pltpu.sync_copy(idx_hbm.at[wid], idx_v)

    def grp(g, carry):
        j0 = g * NBUF
        for b in range(NBUF):
            pltpu.async_copy(
                table_hbm.at[idx_v.at[j0 + b]],
                rows_v.at[pl.ds(b * CHUNK, CHUNK)],
                sem,
            )
        for b in range(NBUF):
            pltpu.make_async_copy(
                table_hbm.at[idx_v.at[j0 + b]],
                rows_v.at[pl.ds(b * CHUNK, CHUNK)],
                sem,
            ).wait()
        pltpu.sync_copy(
            rows_v, out_hbm.at[pl.ds(base + g * GROUP_ROWS, GROUP_ROWS)]
        )
        return carry

    lax.fori_loop(0, GROUPS, grp, 0)


def kernel(x, table):
    idx = x.reshape(NW, G, CHUNK).astype(jnp.int32)
    out = _sc_gather(idx, table)
    return out.reshape(BATCH, FIELDS, HIDDEN)


# speedup vs baseline: 1.5766x; 1.0108x over previous
import functools

import jax
import jax.numpy as jnp
from jax import lax
from jax.experimental import pallas as pl
from jax.experimental.pallas import tpu as pltpu
from jax.experimental.pallas import tpu_sc as plsc

BATCH = 16384
FIELDS = 26
HIDDEN = 32
TOTAL = BATCH * FIELDS

NC = 2
NS = 16
NW = NC * NS
PER_W = TOTAL // NW
CHUNK = 128
G = PER_W // CHUNK
NBUF = 13
GROUPS = G // NBUF
GROUP_ROWS = NBUF * CHUNK

_mesh = plsc.VectorSubcoreMesh(core_axis_name="c", subcore_axis_name="s")


@functools.partial(
    pl.kernel,
    out_type=jax.ShapeDtypeStruct((TOTAL, HIDDEN), jnp.float32),
    mesh=_mesh,
    scratch_types=[
        pltpu.VMEM((G, CHUNK), jnp.int32),
        pltpu.VMEM((2, GROUP_ROWS, HIDDEN), jnp.float32),
        pltpu.SemaphoreType.DMA,
        pltpu.SemaphoreType.DMA,
        pltpu.SemaphoreType.DMA,
    ],
    compiler_params=pltpu.CompilerParams(use_tc_tiling_on_sc=False),
)
def _sc_gather(idx_hbm, table_hbm, out_hbm, idx_v, rows_v, gsem0, gsem1, ssem):
    wid = lax.axis_index("s") * NC + lax.axis_index("c")
    base = wid * PER_W
    pltpu.sync_copy(idx_hbm.at[wid], idx_v)
    gsems = (gsem0, gsem1)

    def fire(g, p):
        for b in range(NBUF):
            pltpu.async_copy(
                table_hbm.at[idx_v.at[g * NBUF + b]],
                rows_v.at[p].at[pl.ds(b * CHUNK, CHUNK)],
                gsems[p],
            )

    def drain(g, p):
        for b in range(NBUF):
            pltpu.make_async_copy(
                table_hbm.at[idx_v.at[g * NBUF + b]],
                rows_v.at[p].at[pl.ds(b * CHUNK, CHUNK)],
                gsems[p],
            ).wait()

    def store(g, p):
        pltpu.async_copy(
            rows_v.at[p], out_hbm.at[pl.ds(base + g * GROUP_ROWS, GROUP_ROWS)],
            ssem,
        )

    def wait_store(g, p):
        pltpu.make_async_copy(
            rows_v.at[p], out_hbm.at[pl.ds(base + g * GROUP_ROWS, GROUP_ROWS)],
            ssem,
        ).wait()

    fire(0, 0)

    def grp2(h, carry):
        for p in range(2):
            g = 2 * h + p
            if p == 0:
                @pl.when(h >= 1)
                def _():
                    wait_store(g - 1, 1)
                fire(g + 1, 1)
            else:
                @pl.when(h < GROUPS // 2 - 1)
                def _():
                    wait_store(g - 1, 0)
                    fire(g + 1, 0)
            drain(g, p)
            store(g, p)
        return carry

    lax.fori_loop(0, GROUPS // 2, grp2, 0)
    wait_store(GROUPS - 2, 0)
    wait_store(GROUPS - 1, 1)


def kernel(x, table):
    idx = x.reshape(NW, G, CHUNK).astype(jnp.int32)
    out = _sc_gather(idx, table)
    return out.reshape(BATCH, FIELDS, HIDDEN)


# speedup vs baseline: 1.5775x; 1.0006x over previous
import functools

import jax
import jax.numpy as jnp
from jax import lax
from jax.experimental import pallas as pl
from jax.experimental.pallas import tpu as pltpu
from jax.experimental.pallas import tpu_sc as plsc

BATCH = 16384
FIELDS = 26
HIDDEN = 32
TOTAL = BATCH * FIELDS

NC = 2
NS = 16
NW = NC * NS
PER_W = TOTAL // NW
CHUNK = 1664
G = PER_W // CHUNK
NBUF = 1
GROUPS = G // NBUF
GROUP_ROWS = NBUF * CHUNK

_mesh = plsc.VectorSubcoreMesh(core_axis_name="c", subcore_axis_name="s")


@functools.partial(
    pl.kernel,
    out_type=jax.ShapeDtypeStruct((TOTAL, HIDDEN), jnp.float32),
    mesh=_mesh,
    scratch_types=[
        pltpu.VMEM((G, CHUNK), jnp.int32),
        pltpu.VMEM((2, GROUP_ROWS, HIDDEN), jnp.float32),
        pltpu.SemaphoreType.DMA,
        pltpu.SemaphoreType.DMA,
        pltpu.SemaphoreType.DMA,
    ],
    compiler_params=pltpu.CompilerParams(use_tc_tiling_on_sc=False),
)
def _sc_gather(idx_hbm, table_hbm, out_hbm, idx_v, rows_v, gsem0, gsem1, ssem):
    wid = lax.axis_index("s") * NC + lax.axis_index("c")
    base = wid * PER_W
    pltpu.sync_copy(idx_hbm.at[wid], idx_v)
    gsems = (gsem0, gsem1)

    def fire(g, p):
        for b in range(NBUF):
            pltpu.async_copy(
                table_hbm.at[idx_v.at[g * NBUF + b]],
                rows_v.at[p].at[pl.ds(b * CHUNK, CHUNK)],
                gsems[p],
            )

    def drain(g, p):
        for b in range(NBUF):
            pltpu.make_async_copy(
                table_hbm.at[idx_v.at[g * NBUF + b]],
                rows_v.at[p].at[pl.ds(b * CHUNK, CHUNK)],
                gsems[p],
            ).wait()

    def store(g, p):
        pltpu.async_copy(
            rows_v.at[p], out_hbm.at[pl.ds(base + g * GROUP_ROWS, GROUP_ROWS)],
            ssem,
        )

    def wait_store(g, p):
        pltpu.make_async_copy(
            rows_v.at[p], out_hbm.at[pl.ds(base + g * GROUP_ROWS, GROUP_ROWS)],
            ssem,
        ).wait()

    fire(0, 0)

    def grp2(h, carry):
        for p in range(2):
            g = 2 * h + p
            if p == 0:
                @pl.when(h >= 1)
                def _():
                    wait_store(g - 1, 1)
                fire(g + 1, 1)
            else:
                @pl.when(h < GROUPS // 2 - 1)
                def _():
                    wait_store(g - 1, 0)
                    fire(g + 1, 0)
            drain(g, p)
            store(g, p)
        return carry

    lax.fori_loop(0, GROUPS // 2, grp2, 0)
    wait_store(GROUPS - 2, 0)
    wait_store(GROUPS - 1, 1)


def kernel(x, table):
    idx = x.reshape(NW, G, CHUNK).astype(jnp.int32)
    out = _sc_gather(idx, table)
    return out.reshape(BATCH, FIELDS, HIDDEN)
